# Initial kernel scaffold; baseline (speedup 1.0000x reference)
import jax
import jax.numpy as jnp
from jax.experimental import pallas as pl


def kernel(x, edge_index, W1, as1, ad1, b1, Wl1, bl1, W2, as2, ad2, b2, Wl2, bl2, W3, as3, ad3, b3, Wl3, bl3):
    raise NotImplementedError("write your pallas kernel here")


# speedup vs baseline: 1.0299x; 1.0299x over previous
import functools
import jax
import jax.numpy as jnp
from jax.experimental import pallas as pl
from jax.experimental.pallas import tpu as pltpu

N = 10000
NPAD = 10240


def _mm_body(x_ref, w_ref, o_ref, acc_ref, *, nk):
    k = pl.program_id(2)

    @pl.when(k == 0)
    def _():
        acc_ref[...] = jnp.zeros_like(acc_ref)

    acc_ref[...] += jnp.dot(x_ref[...], w_ref[...],
                            preferred_element_type=jnp.float32)

    @pl.when(k == nk - 1)
    def _():
        o_ref[...] = acc_ref[...]


def _matmul(x, w, bm=512, bn=512, bk=512):
    m, kdim = x.shape
    _, n = w.shape
    bk = min(bk, kdim)
    bn = min(bn, n)
    nk = kdim // bk
    grid = (m // bm, n // bn, nk)
    return pl.pallas_call(
        functools.partial(_mm_body, nk=nk),
        grid=grid,
        in_specs=[
            pl.BlockSpec((bm, bk), lambda i, j, k: (i, k)),
            pl.BlockSpec((bk, bn), lambda i, j, k: (k, j)),
        ],
        out_specs=pl.BlockSpec((bm, bn), lambda i, j, k: (i, j)),
        scratch_shapes=[pltpu.VMEM((bm, bn), jnp.float32)],
        out_shape=jax.ShapeDtypeStruct((m, n), jnp.float32),
        compiler_params=pltpu.CompilerParams(
            dimension_semantics=("parallel", "parallel", "arbitrary")),
    )(x, w)


def _fold_att(W, att, heads, out_ch):
    return (W.reshape(W.shape[0], heads, out_ch) * att[None]).sum(-1)


def _gat_edges(h, a_src_n, a_dst_n, src, dst, heads, out_ch, concat):
    n = h.shape[0]
    h3 = h.reshape(n, heads, out_ch)
    alpha = a_src_n[src] + a_dst_n[dst]
    alpha = jax.nn.leaky_relu(alpha, negative_slope=0.2)
    ex = jnp.exp(alpha)
    denom = jax.ops.segment_sum(ex, dst, num_segments=n)
    a = ex / (denom[dst] + 1e-16)
    msg = h3[src] * a[:, :, None]
    out = jax.ops.segment_sum(msg, dst, num_segments=n)
    if concat:
        return out.reshape(n, heads * out_ch)
    return out.mean(axis=1)


def kernel(x, edge_index, W1, as1, ad1, b1, Wl1, bl1, W2, as2, ad2, b2, Wl2, bl2, W3, as3, ad3, b3, Wl3, bl3):
    src = edge_index[0]
    dst = edge_index[1]

    def layer(h_in, W, att_s, att_d, b, Wl, bl, heads, out_ch, concat, act):
        hw = W.shape[1]
        ws = _fold_att(W, att_s, heads, out_ch)
        wd = _fold_att(W, att_d, heads, out_ch)
        hpad = 128
        wlw = Wl.shape[1]
        wlp = (wlw + 127) // 128 * 128
        waug = jnp.concatenate([
            W,
            jnp.pad(Wl, ((0, 0), (0, wlp - wlw))),
            jnp.pad(ws, ((0, 0), (0, hpad - heads))),
            jnp.pad(wd, ((0, 0), (0, hpad - heads))),
        ], axis=1)
        haug = _matmul(h_in, waug, bn=256)
        h = haug[:N, :hw]
        lin = haug[:N, hw:hw + wlw]
        a_src_n = haug[:N, hw + wlp:hw + wlp + heads]
        a_dst_n = haug[:N, hw + wlp + hpad:hw + wlp + hpad + heads]
        gat = _gat_edges(h, a_src_n, a_dst_n, src, dst, heads, out_ch, concat)
        out = gat + b + lin + bl
        if act:
            out = jax.nn.elu(out)
        return out

    xp = jnp.pad(x, ((0, NPAD - N), (0, 0)))
    h1 = layer(xp, W1, as1, ad1, b1, Wl1, bl1, 4, 256, True, True)
    h1p = jnp.pad(h1, ((0, NPAD - N), (0, 0)))
    h2 = layer(h1p, W2, as2, ad2, b2, Wl2, bl2, 4, 256, True, True)
    h2p = jnp.pad(h2, ((0, NPAD - N), (0, 0)))
    out = layer(h2p, W3, as3, ad3, b3, Wl3, bl3, 6, 64, False, False)
    return out


# speedup vs baseline: 1.0890x; 1.0574x over previous
import functools
import jax
import jax.numpy as jnp
from jax import lax
from jax.experimental import pallas as pl
from jax.experimental.pallas import tpu as pltpu
from jax.experimental.pallas import tpu_sc as plsc

N = 10000
NPAD = 10240
E = 160000
EP16 = E // 16
EOUT = E // 32
BS = 1000
NP16 = N // 16


def _mm_body(x_ref, w_ref, o_ref, acc_ref, *, nk):
    k = pl.program_id(2)

    @pl.when(k == 0)
    def _():
        acc_ref[...] = jnp.zeros_like(acc_ref)

    acc_ref[...] += jnp.dot(x_ref[...], w_ref[...],
                            preferred_element_type=jnp.float32)

    @pl.when(k == nk - 1)
    def _():
        o_ref[...] = acc_ref[...]


def _matmul(x, w, bm=512, bn=512, bk=512):
    m, kdim = x.shape
    _, n = w.shape
    bk = min(bk, kdim)
    bn = min(bn, n)
    nk = kdim // bk
    grid = (m // bm, n // bn, nk)
    return pl.pallas_call(
        functools.partial(_mm_body, nk=nk),
        grid=grid,
        in_specs=[
            pl.BlockSpec((bm, bk), lambda i, j, k: (i, k)),
            pl.BlockSpec((bk, bn), lambda i, j, k: (k, j)),
        ],
        out_specs=pl.BlockSpec((bm, bn), lambda i, j, k: (i, j)),
        scratch_shapes=[pltpu.VMEM((bm, bn), jnp.float32)],
        out_shape=jax.ShapeDtypeStruct((m, n), jnp.float32),
        compiler_params=pltpu.CompilerParams(
            dimension_semantics=("parallel", "parallel", "arbitrary")),
    )(x, w)


def _fold_att(W, att, heads, out_ch):
    return (W.reshape(W.shape[0], heads, out_ch) * att[None]).sum(-1)


def _sc_softmax_body(asrc_hbm, adst_hbm, src_hbm, dst_hbm, a_hbm,
                     srcb, dstb, asrcb, adstb, exout, denom_sh, sem1, sem2,
                     scale):
    cid = lax.axis_index("c")
    sid = lax.axis_index("s")
    base = sid * EP16
    outbase = base + cid * EOUT

    def zrow(r, _):
        asrcb[r, :] = jnp.zeros((16,), jnp.float32)
        return 0
    lax.fori_loop(0, NP16, zrow, 0)
    pltpu.sync_copy(asrcb.at[pl.ds(0, NP16)], denom_sh.at[pl.ds(sid * NP16, NP16)])
    plsc.subcore_barrier()

    def phase1(bb, _):
        pltpu.sync_copy(src_hbm.at[pl.ds(base + bb * BS, BS)], srcb)
        pltpu.sync_copy(dst_hbm.at[pl.ds(base + bb * BS, BS)], dstb)
        g1 = pltpu.async_copy(asrc_hbm.at[srcb], asrcb, sem1)
        g2 = pltpu.async_copy(adst_hbm.at[dstb], adstb, sem2)
        g1.wait()
        g2.wait()

        def row(r, _):
            s = asrcb[r, :] + adstb[r, :]
            s = jnp.maximum(s, 0.2 * s)
            asrcb[r, :] = jnp.exp(s)
            return 0
        lax.fori_loop(0, BS, row, 0)

        @pl.when(bb // 5 == cid)
        def _():
            def crow(r, _):
                exout[(bb % 5) * BS + r, :] = asrcb[r, :]
                return 0
            lax.fori_loop(0, BS, crow, 0)

        pltpu.sync_copy(asrcb, denom_sh.at[dstb], add=True)
        return 0
    lax.fori_loop(0, EP16 // BS, phase1, 0)
    plsc.subcore_barrier()

    def phase2(o, _):
        pltpu.sync_copy(dst_hbm.at[pl.ds(outbase + o * BS, BS)], dstb)
        pltpu.async_copy(denom_sh.at[dstb], adstb, sem1).wait()

        def row(r, _):
            ex = exout[o * BS + r, :]
            exout[o * BS + r, :] = ex * scale / (adstb[r, :] + 1e-16)
            return 0
        lax.fori_loop(0, BS, row, 0)
        return 0
    lax.fori_loop(0, EOUT // BS, phase2, 0)
    pltpu.sync_copy(exout, a_hbm.at[pl.ds(outbase, EOUT)])


def _sc_softmax(asrc16, adst16, src, dst, scale):
    mesh = plsc.VectorSubcoreMesh(core_axis_name="c", subcore_axis_name="s")
    return pl.kernel(
        functools.partial(_sc_softmax_body, scale=scale),
        out_type=jax.ShapeDtypeStruct((E, 16), jnp.float32),
        mesh=mesh,
        scratch_types=[
            pltpu.VMEM((BS,), jnp.int32),
            pltpu.VMEM((BS,), jnp.int32),
            pltpu.VMEM((BS, 16), jnp.float32),
            pltpu.VMEM((BS, 16), jnp.float32),
            pltpu.VMEM((EOUT, 16), jnp.float32),
            pltpu.VMEM_SHARED((N, 16), jnp.float32),
            pltpu.SemaphoreType.DMA,
            pltpu.SemaphoreType.DMA,
        ],
        compiler_params=pltpu.CompilerParams(use_tc_tiling_on_sc=False),
    )(asrc16, adst16, src, dst)


def _gat_edges(h, a, src, dst, heads, out_ch, concat):
    n = h.shape[0]
    h3 = h.reshape(n, heads, out_ch)
    msg = h3[src] * a[:, :, None]
    out = jax.ops.segment_sum(msg, dst, num_segments=n)
    if concat:
        return out.reshape(n, heads * out_ch)
    return out.sum(axis=1)


def kernel(x, edge_index, W1, as1, ad1, b1, Wl1, bl1, W2, as2, ad2, b2, Wl2, bl2, W3, as3, ad3, b3, Wl3, bl3):
    src = edge_index[0]
    dst = edge_index[1]

    def layer(h_in, W, att_s, att_d, b, Wl, bl, heads, out_ch, concat, act):
        hw = W.shape[1]
        ws = _fold_att(W, att_s, heads, out_ch)
        wd = _fold_att(W, att_d, heads, out_ch)
        hpad = 128
        wlw = Wl.shape[1]
        wlp = (wlw + 127) // 128 * 128
        waug = jnp.concatenate([
            W,
            jnp.pad(Wl, ((0, 0), (0, wlp - wlw))),
            jnp.pad(ws, ((0, 0), (0, hpad - heads))),
            jnp.pad(wd, ((0, 0), (0, hpad - heads))),
        ], axis=1)
        haug = _matmul(h_in, waug, bn=256)
        h = haug[:N, :hw]
        lin = haug[:N, hw:hw + wlw]
        asrc16 = haug[:N, hw + wlp:hw + wlp + 16]
        adst16 = haug[:N, hw + wlp + hpad:hw + wlp + hpad + 16]
        a16 = _sc_softmax(asrc16, adst16, src, dst,
                          scale=1.0 if concat else 1.0 / heads)
        gat = _gat_edges(h, a16[:, :heads], src, dst, heads, out_ch, concat)
        out = gat + b + lin + bl
        if act:
            out = jax.nn.elu(out)
        return out

    xp = jnp.pad(x, ((0, NPAD - N), (0, 0)))
    h1 = layer(xp, W1, as1, ad1, b1, Wl1, bl1, 4, 256, True, True)
    h1p = jnp.pad(h1, ((0, NPAD - N), (0, 0)))
    h2 = layer(h1p, W2, as2, ad2, b2, Wl2, bl2, 4, 256, True, True)
    h2p = jnp.pad(h2, ((0, NPAD - N), (0, 0)))
    out = layer(h2p, W3, as3, ad3, b3, Wl3, bl3, 6, 64, False, False)
    return out


# speedup vs baseline: 13.5727x; 12.4636x over previous
import functools
import jax
import jax.numpy as jnp
from jax import lax
from jax.experimental import pallas as pl
from jax.experimental.pallas import tpu as pltpu
from jax.experimental.pallas import tpu_sc as plsc

N = 10000
NPAD = 10240
E = 160000
EP16 = E // 16
EOUT = E // 32
BS = 1000
NP16 = N // 16


def _mm_body(x_ref, w_ref, o_ref, acc_ref, *, nk):
    k = pl.program_id(2)

    @pl.when(k == 0)
    def _():
        acc_ref[...] = jnp.zeros_like(acc_ref)

    acc_ref[...] += jnp.dot(x_ref[...], w_ref[...],
                            preferred_element_type=jnp.float32)

    @pl.when(k == nk - 1)
    def _():
        o_ref[...] = acc_ref[...]


def _matmul(x, w, bm=512, bn=512, bk=512):
    m, kdim = x.shape
    _, n = w.shape
    bk = min(bk, kdim)
    bn = min(bn, n)
    nk = kdim // bk
    grid = (m // bm, n // bn, nk)
    return pl.pallas_call(
        functools.partial(_mm_body, nk=nk),
        grid=grid,
        in_specs=[
            pl.BlockSpec((bm, bk), lambda i, j, k: (i, k)),
            pl.BlockSpec((bk, bn), lambda i, j, k: (k, j)),
        ],
        out_specs=pl.BlockSpec((bm, bn), lambda i, j, k: (i, j)),
        scratch_shapes=[pltpu.VMEM((bm, bn), jnp.float32)],
        out_shape=jax.ShapeDtypeStruct((m, n), jnp.float32),
        compiler_params=pltpu.CompilerParams(
            dimension_semantics=("parallel", "parallel", "arbitrary")),
    )(x, w)


def _fold_att(W, att, heads, out_ch):
    return (W.reshape(W.shape[0], heads, out_ch) * att[None]).sum(-1)


def _sc_softmax_body(asrc_hbm, adst_hbm, src_hbm, dst_hbm, a_hbm,
                     srcb, dstb, asrcb, adstb, exout, denom_sh, sem1, sem2,
                     scale):
    cid = lax.axis_index("c")
    sid = lax.axis_index("s")
    base = sid * EP16
    outbase = base + cid * EOUT

    def zrow(r, _):
        asrcb[r, :] = jnp.zeros((16,), jnp.float32)
        return 0
    lax.fori_loop(0, NP16, zrow, 0)
    pltpu.sync_copy(asrcb.at[pl.ds(0, NP16)], denom_sh.at[pl.ds(sid * NP16, NP16)])
    plsc.subcore_barrier()

    def phase1(bb, _):
        pltpu.sync_copy(src_hbm.at[pl.ds(base + bb * BS, BS)], srcb)
        pltpu.sync_copy(dst_hbm.at[pl.ds(base + bb * BS, BS)], dstb)
        g1 = pltpu.async_copy(asrc_hbm.at[srcb], asrcb, sem1)
        g2 = pltpu.async_copy(adst_hbm.at[dstb], adstb, sem2)
        g1.wait()
        g2.wait()

        def row(r, _):
            s = asrcb[r, :] + adstb[r, :]
            s = jnp.maximum(s, 0.2 * s)
            asrcb[r, :] = jnp.exp(s)
            return 0
        lax.fori_loop(0, BS, row, 0)

        @pl.when(bb // 5 == cid)
        def _():
            def crow(r, _):
                exout[(bb % 5) * BS + r, :] = asrcb[r, :]
                return 0
            lax.fori_loop(0, BS, crow, 0)

        pltpu.sync_copy(asrcb, denom_sh.at[dstb], add=True)
        return 0
    lax.fori_loop(0, EP16 // BS, phase1, 0)
    plsc.subcore_barrier()

    def phase2(o, _):
        pltpu.sync_copy(dst_hbm.at[pl.ds(outbase + o * BS, BS)], dstb)
        pltpu.async_copy(denom_sh.at[dstb], adstb, sem1).wait()

        def row(r, _):
            ex = exout[o * BS + r, :]
            exout[o * BS + r, :] = ex * scale / (adstb[r, :] + 1e-16)
            return 0
        lax.fori_loop(0, BS, row, 0)
        return 0
    lax.fori_loop(0, EOUT // BS, phase2, 0)
    pltpu.sync_copy(exout, a_hbm.at[pl.ds(outbase, EOUT)])


def _sc_softmax(asrc16, adst16, src, dst, scale):
    mesh = plsc.VectorSubcoreMesh(core_axis_name="c", subcore_axis_name="s")
    return pl.kernel(
        functools.partial(_sc_softmax_body, scale=scale),
        out_type=jax.ShapeDtypeStruct((E, 16), jnp.float32),
        mesh=mesh,
        scratch_types=[
            pltpu.VMEM((BS,), jnp.int32),
            pltpu.VMEM((BS,), jnp.int32),
            pltpu.VMEM((BS, 16), jnp.float32),
            pltpu.VMEM((BS, 16), jnp.float32),
            pltpu.VMEM((EOUT, 16), jnp.float32),
            pltpu.VMEM_SHARED((N, 16), jnp.float32),
            pltpu.SemaphoreType.DMA,
            pltpu.SemaphoreType.DMA,
        ],
        compiler_params=pltpu.CompilerParams(use_tc_tiling_on_sc=False, needs_layout_passes=False),
    )(asrc16, adst16, src, dst)


RCH = 512
NCH = 20
ACC = RCH + 8
BA = 32
RPT = RCH // 16


def _sc_agg_body(h_hbm, a_hbm, src_hbm, dst_hbm, agg_hbm,
                 csrc, cdst, ce, srcb, dstb, hrows, arows, srows, bidx,
                 accum_sh, sem1, sem2,
                 *, hw, heads, concat, pair):
    cid = lax.axis_index("c")
    sid = lax.axis_index("s")
    ebase = sid * EP16
    wout = hw if concat else hw // heads
    C = hw // heads
    nj = C // 16
    zb = hrows if concat else srows
    base = (2 * pair + cid) * RCH
    obase = cid * RCH

    pltpu.sync_copy(src_hbm.at[pl.ds(sid * (EP16 // 16), EP16 // 16)], srcb)
    pltpu.sync_copy(dst_hbm.at[pl.ds(sid * (EP16 // 16), EP16 // 16)], dstb)

    def zr(r, _):
        for j in range(wout // 16):
            zb[r, pl.ds(j * 16, 16)] = jnp.zeros((16,), jnp.float32)
        return 0
    lax.fori_loop(0, min(BA, RPT), zr, 0)
    pltpu.sync_copy(zb.at[pl.ds(0, RPT)], accum_sh.at[pl.ds(sid * RPT, RPT)])
    plsc.subcore_barrier()

    def blk(i, p):
        dstv = dstb[i, :]
        srcv = srcb[i, :]
        rel = dstv - base
        mask = (rel >= 0) & (rel < RCH)
        mi = jnp.where(mask, 1, 0)
        pos = p + plsc.cumsum(mi) - 1
        eid = ebase + i * 16 + lax.iota(jnp.int32, 16)
        plsc.store_scatter(csrc, [pos], srcv, mask=mask)
        plsc.store_scatter(cdst, [pos], rel, mask=mask)
        plsc.store_scatter(ce, [pos], eid, mask=mask)
        return p + jnp.sum(mi)
    m = lax.fori_loop(0, EP16 // 16, blk, 0)

    zv = jnp.zeros((16,), jnp.int32)
    rv = jnp.full((16,), RCH, jnp.int32)
    for j in range(3):
        pos = m + j * 16 + lax.iota(jnp.int32, 16)
        plsc.store_scatter(csrc, [pos], zv)
        plsc.store_scatter(cdst, [pos], rv)
        plsc.store_scatter(ce, [pos], zv)

    nb = (m + BA - 1) // BA

    def rowit(t, _):
        b = t // BA
        r = lax.rem(t, BA)

        @pl.when(r == 0)
        def _():
            for j in range(BA // 16):
                idx = b * BA + j * 16 + lax.iota(jnp.int32, 16)
                bidx[pl.ds(j * 16, 16)] = plsc.load_gather(cdst, [idx])
            g1 = pltpu.async_copy(
                h_hbm.at[csrc.at[pl.ds(b * BA, BA)]], hrows, sem1)
            g2 = pltpu.async_copy(
                a_hbm.at[ce.at[pl.ds(b * BA, BA)]], arows, sem2)
            g1.wait()
            g2.wait()

        av = arows[r, :]
        if concat:
            for k in range(heads):
                am = av[k]
                for j in range(nj):
                    col = k * C + j * 16
                    hrows[r, pl.ds(col, 16)] = hrows[r, pl.ds(col, 16)] * am
        else:
            for j in range(nj):
                acc = jnp.zeros((16,), jnp.float32)
                for k in range(heads):
                    acc = acc + hrows[r, pl.ds(k * C + j * 16, 16)] * av[k]
                srows[r, pl.ds(j * 16, 16)] = acc

        @pl.when(r == BA - 1)
        def _():
            if concat:
                pltpu.sync_copy(hrows, accum_sh.at[bidx], add=True)
            else:
                pltpu.sync_copy(srows, accum_sh.at[bidx], add=True)
        return 0
    lax.fori_loop(0, nb * BA, rowit, 0)
    plsc.subcore_barrier()

    pltpu.sync_copy(accum_sh.at[pl.ds(sid * RPT, RPT)],
                    agg_hbm.at[pl.ds(obase + sid * RPT, RPT)])


def _sc_aggregate(h, a16, src2, dst2, hw, heads, concat, pair):
    wout = hw if concat else hw // heads
    mesh = plsc.VectorSubcoreMesh(core_axis_name="c", subcore_axis_name="s")
    return pl.kernel(
        functools.partial(_sc_agg_body, hw=hw, heads=heads, concat=concat,
                          pair=pair),
        out_type=jax.ShapeDtypeStruct((2 * RCH, wout), jnp.float32),
        mesh=mesh,
        scratch_types=[
            pltpu.VMEM((EP16 + 64,), jnp.int32),
            pltpu.VMEM((EP16 + 64,), jnp.int32),
            pltpu.VMEM((EP16 + 64,), jnp.int32),
            pltpu.VMEM((EP16 // 16, 16), jnp.int32),
            pltpu.VMEM((EP16 // 16, 16), jnp.int32),
            pltpu.VMEM((BA, hw), jnp.float32),
            pltpu.VMEM((BA, 16), jnp.float32),
            pltpu.VMEM((16, 16) if concat else (BA, wout), jnp.float32),
            pltpu.VMEM((BA,), jnp.int32),
            pltpu.VMEM_SHARED((ACC, wout), jnp.float32),
            pltpu.SemaphoreType.DMA,
            pltpu.SemaphoreType.DMA,
        ],
        compiler_params=pltpu.CompilerParams(use_tc_tiling_on_sc=False, needs_layout_passes=False),
    )(h, a16, src2, dst2)


def _combine(agg, lin, bias2d, act, wlw):
    m, wout = agg.shape
    wlp = lin.shape[1]
    bm = 512

    def body(a_ref, l_ref, b_ref, o_ref):
        t = a_ref[...] + l_ref[...][:, :wlw] + b_ref[...][0:1, :]
        if act:
            t = jnp.where(t > 0, t, jnp.exp(jnp.minimum(t, 0.0)) - 1.0)
        o_ref[...] = t

    return pl.pallas_call(
        body,
        grid=(m // bm,),
        in_specs=[
            pl.BlockSpec((bm, wout), lambda i: (i, 0)),
            pl.BlockSpec((bm, wlp), lambda i: (i, 0)),
            pl.BlockSpec((8, wout), lambda i: (0, 0)),
        ],
        out_specs=pl.BlockSpec((bm, wout), lambda i: (i, 0)),
        out_shape=jax.ShapeDtypeStruct((m, wout), jnp.float32),
    )(agg, lin, bias2d)


def kernel(x, edge_index, W1, as1, ad1, b1, Wl1, bl1, W2, as2, ad2, b2, Wl2, bl2, W3, as3, ad3, b3, Wl3, bl3):
    src = edge_index[0]
    dst = edge_index[1]

    def layer(h_in, W, att_s, att_d, b, Wl, bl, heads, out_ch, concat, act):
        hw = W.shape[1]
        wout = hw if concat else out_ch
        h = _matmul(h_in, W, bn=512 if hw % 512 == 0 else 128)
        wlw = Wl.shape[1]
        wlp = max(128, wlw)
        lin = _matmul(h_in, jnp.pad(Wl, ((0, 0), (0, wlp - wlw))),
                      bn=512 if wlp % 512 == 0 else 128)
        ws = _fold_att(W, att_s, heads, out_ch)
        wd = _fold_att(W, att_d, heads, out_ch)
        wsd = jnp.concatenate([
            jnp.pad(ws, ((0, 0), (0, 16 - heads))),
            jnp.pad(wd, ((0, 0), (0, 16 - heads))),
            jnp.zeros((W.shape[0], 96), jnp.float32),
        ], axis=1)
        sc = _matmul(h_in, wsd, bn=128)
        asrc16 = sc[:N, 0:16]
        adst16 = sc[:N, 16:32]
        a16 = _sc_softmax(asrc16, adst16, src, dst,
                          scale=1.0 if concat else 1.0 / heads)
        src2 = src.reshape(E // 16, 16)
        dst2 = dst.reshape(E // 16, 16)
        agg = jnp.concatenate(
            [_sc_aggregate(h, a16, src2, dst2, hw, heads, concat, pair)
             for pair in range(NCH // 2)], axis=0)
        bias2d = jnp.broadcast_to(b[None, :] + bl[None, :], (8, wout))
        return _combine(agg, lin, bias2d, act, wlw)

    xp = jnp.pad(x, ((0, NPAD - N), (0, 0)))
    h1 = layer(xp, W1, as1, ad1, b1, Wl1, bl1, 4, 256, True, True)
    h2 = layer(h1, W2, as2, ad2, b2, Wl2, bl2, 4, 256, True, True)
    out = layer(h2, W3, as3, ad3, b3, Wl3, bl3, 6, 64, False, False)
    return out[:N]


# speedup vs baseline: 14.2109x; 1.0470x over previous
import functools
import jax
import jax.numpy as jnp
from jax import lax
from jax.experimental import pallas as pl
from jax.experimental.pallas import tpu as pltpu
from jax.experimental.pallas import tpu_sc as plsc

N = 10000
NPAD = 10240
E = 160000
EP16 = E // 16
EOUT = E // 32
BS = 1000
NP16 = N // 16


def _mm_body(x_ref, w_ref, o_ref, acc_ref, *, nk):
    k = pl.program_id(2)

    @pl.when(k == 0)
    def _():
        acc_ref[...] = jnp.zeros_like(acc_ref)

    acc_ref[...] += jnp.dot(x_ref[...], w_ref[...],
                            preferred_element_type=jnp.float32)

    @pl.when(k == nk - 1)
    def _():
        o_ref[...] = acc_ref[...]


def _matmul(x, w, bm=512, bn=512, bk=512):
    m, kdim = x.shape
    _, n = w.shape
    bk = min(bk, kdim)
    bn = min(bn, n)
    nk = kdim // bk
    grid = (m // bm, n // bn, nk)
    return pl.pallas_call(
        functools.partial(_mm_body, nk=nk),
        grid=grid,
        in_specs=[
            pl.BlockSpec((bm, bk), lambda i, j, k: (i, k)),
            pl.BlockSpec((bk, bn), lambda i, j, k: (k, j)),
        ],
        out_specs=pl.BlockSpec((bm, bn), lambda i, j, k: (i, j)),
        scratch_shapes=[pltpu.VMEM((bm, bn), jnp.float32)],
        out_shape=jax.ShapeDtypeStruct((m, n), jnp.float32),
        compiler_params=pltpu.CompilerParams(
            dimension_semantics=("parallel", "parallel", "arbitrary")),
    )(x, w)


def _fold_att(W, att, heads, out_ch):
    return (W.reshape(W.shape[0], heads, out_ch) * att[None]).sum(-1)


def _sc_softmax_body(asrc_hbm, adst_hbm, src_hbm, dst_hbm, a_hbm,
                     srcb, dstb, asrcb, adstb, exout, denom_sh, sem1, sem2,
                     scale):
    cid = lax.axis_index("c")
    sid = lax.axis_index("s")
    base = sid * EP16
    outbase = base + cid * EOUT

    def zrow(r, _):
        asrcb[r, :] = jnp.zeros((16,), jnp.float32)
        return 0
    lax.fori_loop(0, NP16, zrow, 0)
    pltpu.sync_copy(asrcb.at[pl.ds(0, NP16)], denom_sh.at[pl.ds(sid * NP16, NP16)])
    plsc.subcore_barrier()

    def phase1(bb, _):
        pltpu.sync_copy(src_hbm.at[pl.ds(base + bb * BS, BS)], srcb)
        pltpu.sync_copy(dst_hbm.at[pl.ds(base + bb * BS, BS)], dstb)
        g1 = pltpu.async_copy(asrc_hbm.at[srcb], asrcb, sem1)
        g2 = pltpu.async_copy(adst_hbm.at[dstb], adstb, sem2)
        g1.wait()
        g2.wait()

        def row(r, _):
            s = asrcb[r, :] + adstb[r, :]
            s = jnp.maximum(s, 0.2 * s)
            asrcb[r, :] = jnp.exp(s)
            return 0
        lax.fori_loop(0, BS, row, 0)

        @pl.when(bb // 5 == cid)
        def _():
            def crow(r, _):
                exout[(bb % 5) * BS + r, :] = asrcb[r, :]
                return 0
            lax.fori_loop(0, BS, crow, 0)

        pltpu.sync_copy(asrcb, denom_sh.at[dstb], add=True)
        return 0
    lax.fori_loop(0, EP16 // BS, phase1, 0)
    plsc.subcore_barrier()

    def phase2(o, _):
        pltpu.sync_copy(dst_hbm.at[pl.ds(outbase + o * BS, BS)], dstb)
        pltpu.async_copy(denom_sh.at[dstb], adstb, sem1).wait()

        def row(r, _):
            ex = exout[o * BS + r, :]
            exout[o * BS + r, :] = ex * scale / (adstb[r, :] + 1e-16)
            return 0
        lax.fori_loop(0, BS, row, 0)
        return 0
    lax.fori_loop(0, EOUT // BS, phase2, 0)
    pltpu.sync_copy(exout, a_hbm.at[pl.ds(outbase, EOUT)])


def _sc_softmax(asrc16, adst16, src, dst, scale):
    mesh = plsc.VectorSubcoreMesh(core_axis_name="c", subcore_axis_name="s")
    return pl.kernel(
        functools.partial(_sc_softmax_body, scale=scale),
        out_type=jax.ShapeDtypeStruct((E, 16), jnp.float32),
        mesh=mesh,
        scratch_types=[
            pltpu.VMEM((BS,), jnp.int32),
            pltpu.VMEM((BS,), jnp.int32),
            pltpu.VMEM((BS, 16), jnp.float32),
            pltpu.VMEM((BS, 16), jnp.float32),
            pltpu.VMEM((EOUT, 16), jnp.float32),
            pltpu.VMEM_SHARED((N, 16), jnp.float32),
            pltpu.SemaphoreType.DMA,
            pltpu.SemaphoreType.DMA,
        ],
        compiler_params=pltpu.CompilerParams(use_tc_tiling_on_sc=False, needs_layout_passes=False),
    )(asrc16, adst16, src, dst)


RCH = 512
NCH = 20
ACC = RCH + 8
BA = 32
RPT = RCH // 16


def _sc_agg_body(h_hbm, a_hbm, src_hbm, dst_hbm, agg_hbm,
                 csrc, cdst, ce, srcb, dstb, hrows, arows, srows, bidx,
                 accum_sh, sem1, sem2,
                 *, hw, heads, concat, pair0, npairs):
    cid = lax.axis_index("c")
    sid = lax.axis_index("s")
    ebase = sid * EP16
    wout = hw if concat else hw // heads
    C = hw // heads
    nj = C // 16
    zb = hrows if concat else srows

    pltpu.sync_copy(src_hbm.at[pl.ds(sid * (EP16 // 16), EP16 // 16)], srcb)
    pltpu.sync_copy(dst_hbm.at[pl.ds(sid * (EP16 // 16), EP16 // 16)], dstb)

    for q in range(npairs):
        _sc_agg_pair(h_hbm, a_hbm, agg_hbm, csrc, cdst, ce, srcb, dstb,
                     hrows, arows, srows, bidx, accum_sh, sem1, sem2,
                     cid, sid, ebase, wout, C, nj, zb,
                     base=(2 * (pair0 + q) + cid) * RCH,
                     obase=(2 * q + cid) * RCH,
                     heads=heads, concat=concat)


def _sc_agg_pair(h_hbm, a_hbm, agg_hbm, csrc, cdst, ce, srcb, dstb,
                 hrows, arows, srows, bidx, accum_sh, sem1, sem2,
                 cid, sid, ebase, wout, C, nj, zb,
                 *, base, obase, heads, concat):
    def zr(r, _):
        for j in range(wout // 16):
            zb[r, pl.ds(j * 16, 16)] = jnp.zeros((16,), jnp.float32)
        return 0
    lax.fori_loop(0, min(BA, RPT), zr, 0)
    pltpu.sync_copy(zb.at[pl.ds(0, RPT)], accum_sh.at[pl.ds(sid * RPT, RPT)])
    plsc.subcore_barrier()

    def blk(i, p):
        dstv = dstb[i, :]
        srcv = srcb[i, :]
        rel = dstv - base
        mask = (rel >= 0) & (rel < RCH)
        mi = jnp.where(mask, 1, 0)
        pos = p + plsc.cumsum(mi) - 1
        eid = ebase + i * 16 + lax.iota(jnp.int32, 16)
        plsc.store_scatter(csrc, [pos], srcv, mask=mask)
        plsc.store_scatter(cdst, [pos], rel, mask=mask)
        plsc.store_scatter(ce, [pos], eid, mask=mask)
        return p + jnp.sum(mi)
    m = lax.fori_loop(0, EP16 // 16, blk, 0)

    zv = jnp.zeros((16,), jnp.int32)
    rv = jnp.full((16,), RCH, jnp.int32)
    for j in range(3):
        pos = m + j * 16 + lax.iota(jnp.int32, 16)
        plsc.store_scatter(csrc, [pos], zv)
        plsc.store_scatter(cdst, [pos], rv)
        plsc.store_scatter(ce, [pos], zv)

    nb = (m + BA - 1) // BA

    def rowit(t, _):
        b = t // BA
        r = lax.rem(t, BA)

        @pl.when(r == 0)
        def _():
            for j in range(BA // 16):
                idx = b * BA + j * 16 + lax.iota(jnp.int32, 16)
                bidx[pl.ds(j * 16, 16)] = plsc.load_gather(cdst, [idx])
            g1 = pltpu.async_copy(
                h_hbm.at[csrc.at[pl.ds(b * BA, BA)]], hrows, sem1)
            g2 = pltpu.async_copy(
                a_hbm.at[ce.at[pl.ds(b * BA, BA)]], arows, sem2)
            g1.wait()
            g2.wait()

        av = arows[r, :]
        if concat:
            for k in range(heads):
                am = av[k]
                for j in range(nj):
                    col = k * C + j * 16
                    hrows[r, pl.ds(col, 16)] = hrows[r, pl.ds(col, 16)] * am
        else:
            for j in range(nj):
                acc = jnp.zeros((16,), jnp.float32)
                for k in range(heads):
                    acc = acc + hrows[r, pl.ds(k * C + j * 16, 16)] * av[k]
                srows[r, pl.ds(j * 16, 16)] = acc

        @pl.when(r == BA - 1)
        def _():
            if concat:
                pltpu.sync_copy(hrows, accum_sh.at[bidx], add=True)
            else:
                pltpu.sync_copy(srows, accum_sh.at[bidx], add=True)
        return 0
    lax.fori_loop(0, nb * BA, rowit, 0)
    plsc.subcore_barrier()

    pltpu.sync_copy(accum_sh.at[pl.ds(sid * RPT, RPT)],
                    agg_hbm.at[pl.ds(obase + sid * RPT, RPT)])


def _sc_aggregate(h, a16, src2, dst2, hw, heads, concat, pair0, npairs):
    wout = hw if concat else hw // heads
    mesh = plsc.VectorSubcoreMesh(core_axis_name="c", subcore_axis_name="s")
    return pl.kernel(
        functools.partial(_sc_agg_body, hw=hw, heads=heads, concat=concat,
                          pair0=pair0, npairs=npairs),
        out_type=jax.ShapeDtypeStruct((2 * npairs * RCH, wout), jnp.float32),
        mesh=mesh,
        scratch_types=[
            pltpu.VMEM((EP16 + 64,), jnp.int32),
            pltpu.VMEM((EP16 + 64,), jnp.int32),
            pltpu.VMEM((EP16 + 64,), jnp.int32),
            pltpu.VMEM((EP16 // 16, 16), jnp.int32),
            pltpu.VMEM((EP16 // 16, 16), jnp.int32),
            pltpu.VMEM((BA, hw), jnp.float32),
            pltpu.VMEM((BA, 16), jnp.float32),
            pltpu.VMEM((16, 16) if concat else (BA, wout), jnp.float32),
            pltpu.VMEM((BA,), jnp.int32),
            pltpu.VMEM_SHARED((ACC, wout), jnp.float32),
            pltpu.SemaphoreType.DMA,
            pltpu.SemaphoreType.DMA,
        ],
        compiler_params=pltpu.CompilerParams(use_tc_tiling_on_sc=False, needs_layout_passes=False),
    )(h, a16, src2, dst2)


def _combine(agg, lin, bias2d, act, wlw):
    m, wout = agg.shape
    wlp = lin.shape[1]
    bm = 512

    def body(a_ref, l_ref, b_ref, o_ref):
        t = a_ref[...] + l_ref[...][:, :wlw] + b_ref[...][0:1, :]
        if act:
            t = jnp.where(t > 0, t, jnp.exp(jnp.minimum(t, 0.0)) - 1.0)
        o_ref[...] = t

    return pl.pallas_call(
        body,
        grid=(m // bm,),
        in_specs=[
            pl.BlockSpec((bm, wout), lambda i: (i, 0)),
            pl.BlockSpec((bm, wlp), lambda i: (i, 0)),
            pl.BlockSpec((8, wout), lambda i: (0, 0)),
        ],
        out_specs=pl.BlockSpec((bm, wout), lambda i: (i, 0)),
        out_shape=jax.ShapeDtypeStruct((m, wout), jnp.float32),
    )(agg, lin, bias2d)


def kernel(x, edge_index, W1, as1, ad1, b1, Wl1, bl1, W2, as2, ad2, b2, Wl2, bl2, W3, as3, ad3, b3, Wl3, bl3):
    src = edge_index[0]
    dst = edge_index[1]

    def layer(h_in, W, att_s, att_d, b, Wl, bl, heads, out_ch, concat, act):
        hw = W.shape[1]
        wout = hw if concat else out_ch
        h = _matmul(h_in, W, bn=512 if hw % 512 == 0 else 128)
        wlw = Wl.shape[1]
        wlp = max(128, wlw)
        lin = _matmul(h_in, jnp.pad(Wl, ((0, 0), (0, wlp - wlw))),
                      bn=512 if wlp % 512 == 0 else 128)
        ws = _fold_att(W, att_s, heads, out_ch)
        wd = _fold_att(W, att_d, heads, out_ch)
        wsd = jnp.concatenate([
            jnp.pad(ws, ((0, 0), (0, 16 - heads))),
            jnp.pad(wd, ((0, 0), (0, 16 - heads))),
            jnp.zeros((W.shape[0], 96), jnp.float32),
        ], axis=1)
        sc = _matmul(h_in, wsd, bn=128)
        asrc16 = sc[:N, 0:16]
        adst16 = sc[:N, 16:32]
        a16 = _sc_softmax(asrc16, adst16, src, dst,
                          scale=1.0 if concat else 1.0 / heads)
        src2 = src.reshape(E // 16, 16)
        dst2 = dst.reshape(E // 16, 16)
        npp = 5
        agg = jnp.concatenate(
            [_sc_aggregate(h, a16, src2, dst2, hw, heads, concat, p0, npp)
             for p0 in range(0, NCH // 2, npp)], axis=0)
        bias2d = jnp.broadcast_to(b[None, :] + bl[None, :], (8, wout))
        return _combine(agg, lin, bias2d, act, wlw)

    xp = jnp.pad(x, ((0, NPAD - N), (0, 0)))
    h1 = layer(xp, W1, as1, ad1, b1, Wl1, bl1, 4, 256, True, True)
    h2 = layer(h1, W2, as2, ad2, b2, Wl2, bl2, 4, 256, True, True)
    out = layer(h2, W3, as3, ad3, b3, Wl3, bl3, 6, 64, False, False)
    return out[:N]
